# Initial kernel scaffold; baseline (speedup 1.0000x reference)
import jax
import jax.numpy as jnp
from jax.experimental import pallas as pl


def kernel(params, subject_idx):
    raise NotImplementedError("write your pallas kernel here")


# speedup vs baseline: 1.5311x; 1.5311x over previous
import jax
import jax.numpy as jnp
from jax import lax
from jax.experimental import pallas as pl
from jax.experimental.pallas import tpu as pltpu
from jax.experimental.pallas import tpu_sc as plsc

N_ROWS = 4_000_000
NUM_SUBJ = 64
LANES = 16
CHUNK = 4096
GROUPS = CHUNK // LANES
NUM_TILES = 32
NUM_CHUNKS = (N_ROWS + CHUNK - 1) // CHUNK
TRIPS = (NUM_CHUNKS + NUM_TILES - 1) // NUM_TILES
ACC_WORDS = LANES * 4 * NUM_SUBJ


def _sc_segment_partials(params_ref, sidx_ref, out_ref, par_buf, idx_buf,
                         acc, sem_p, sem_i):
    wid = lax.axis_index("s") * 2 + lax.axis_index("c")
    lane = lax.iota(jnp.int32, LANES)
    ones = jnp.ones((LANES,), jnp.float32)

    def zero_body(i, _):
        acc[pl.ds(i * LANES, LANES)] = jnp.zeros((LANES,), jnp.float32)
        return 0
    lax.fori_loop(0, ACC_WORDS // LANES, zero_body, 0)

    def trip_body(k, _):
        ci = wid + k * NUM_TILES
        nominal = ci * CHUNK
        start = jnp.minimum(nominal, N_ROWS - CHUNK)
        delta = nominal - start
        cp_p = pltpu.async_copy(params_ref.at[pl.ds(start * 3, CHUNK * 3)],
                                par_buf, sem_p)
        cp_i = pltpu.async_copy(sidx_ref.at[pl.ds(start, CHUNK)], idx_buf,
                                sem_i)
        cp_p.wait()
        cp_i.wait()

        lane3 = lane * 3

        def group_body(g, _):
            base = g * LANES
            rows = base + lane
            msk = rows >= delta
            subj = idx_buf[pl.ds(base, LANES)]
            addr = lane * 256 + subj
            fbase = base * 3 + lane3
            for c in range(3):
                v = plsc.load_gather(par_buf, [fbase + c])
                plsc.addupdate_scatter(acc, [addr + c * NUM_SUBJ], v,
                                       mask=msk)
            plsc.addupdate_scatter(acc, [addr + 3 * NUM_SUBJ], ones, mask=msk)
            return 0
        lax.fori_loop(0, GROUPS, group_body, 0)
        return 0

    lax.fori_loop(0, TRIPS, trip_body, 0)
    pltpu.sync_copy(acc, out_ref.at[wid])


def _tc_stats(x_ref, out_ref):
    x = x_ref[...]
    t = jnp.sum(x, axis=0, keepdims=True)
    counts = t[:, 3 * NUM_SUBJ:4 * NUM_SUBJ]
    present = (counts > 0).astype(jnp.float32)
    m = jnp.sum(present)
    safe_counts = jnp.maximum(counts, 1.0)
    targets = (400.0, 25.0, 0.05)
    loss = jnp.float32(0.0)
    for c in range(3):
        s_c = t[:, c * NUM_SUBJ:(c + 1) * NUM_SUBJ]
        mean_c = s_c / safe_counts
        mom_c = jnp.sum(mean_c * present) / m
        var_c = jnp.sum(present * (mean_c - mom_c) ** 2) / (m - 1.0)
        std_c = jnp.sqrt(var_c)
        loss = loss + jnp.maximum(targets[c] - std_c, 0.0)
    loss = jnp.where(m < 2.0, jnp.float32(0.0), loss)
    out_ref[0, 0] = loss


def kernel(params, subject_idx):
    params = params.astype(jnp.float32).reshape(-1)
    sidx = subject_idx.astype(jnp.int32)

    mesh = plsc.VectorSubcoreMesh(core_axis_name="c", subcore_axis_name="s")
    partials = pl.kernel(
        _sc_segment_partials,
        out_type=jax.ShapeDtypeStruct((NUM_TILES, ACC_WORDS), jnp.float32),
        mesh=mesh,
        compiler_params=pltpu.CompilerParams(needs_layout_passes=False),
        scratch_types=[
            pltpu.VMEM((CHUNK * 3,), jnp.float32),
            pltpu.VMEM((CHUNK,), jnp.int32),
            pltpu.VMEM((ACC_WORDS,), jnp.float32),
            pltpu.SemaphoreType.DMA,
            pltpu.SemaphoreType.DMA,
        ],
    )(params, sidx)

    stacked = partials.reshape(NUM_TILES * LANES, 4 * NUM_SUBJ)
    loss = pl.pallas_call(
        _tc_stats,
        out_shape=jax.ShapeDtypeStruct((1, 1), jnp.float32),
        out_specs=pl.BlockSpec(memory_space=pltpu.SMEM),
    )(stacked)
    return loss[0, 0]


# speedup vs baseline: 9.0875x; 5.9354x over previous
import jax
import jax.numpy as jnp
from jax import lax
from jax.experimental import pallas as pl
from jax.experimental.pallas import tpu as pltpu
from jax.experimental.pallas import tpu_sc as plsc

N_ROWS = 4_000_000
NUM_SUBJ = 64
LANES = 16
CHUNK = 256
GROUPS = CHUNK // LANES
NUM_TILES = 32
NUM_CHUNKS = N_ROWS // CHUNK
TRIPS = (NUM_CHUNKS + NUM_TILES - 1) // NUM_TILES
ACC_WORDS = LANES * 4 * NUM_SUBJ


def _sc_segment_partials(params_ref, sidx_ref, out_ref, par_buf, idx_buf,
                         rowid_buf, acc, sem_p, sem_i):
    wid = lax.axis_index("s") * 2 + lax.axis_index("c")
    lane = lax.iota(jnp.int32, LANES)
    ones = jnp.ones((LANES,), jnp.float32)

    def zero_body(i, _):
        acc[pl.ds(i * LANES, LANES)] = jnp.zeros((LANES,), jnp.float32)
        return 0
    lax.fori_loop(0, ACC_WORDS // LANES, zero_body, 0)

    def trip_body(k, _):
        ci = wid + k * NUM_TILES
        nominal = ci * CHUNK
        start = jnp.minimum(nominal, N_ROWS - CHUNK)
        delta = nominal - start
        cp_p = pltpu.async_copy(params_ref.at[pl.ds(start, CHUNK)], par_buf,
                                sem_p)
        cp_i = pltpu.async_copy(sidx_ref.at[pl.ds(start, CHUNK)], idx_buf,
                                sem_i)
        cp_p.wait()
        cp_i.wait()

        def group_body(g, _):
            base = g * LANES
            rows = base + lane
            msk = rows >= delta
            subj = idx_buf[pl.ds(base, LANES)]
            addr = lane * 256 + subj
            for c in range(3):
                col = jnp.full((LANES,), c, jnp.int32)
                v = plsc.load_gather(par_buf, [rows, col])
                plsc.addupdate_scatter(acc, [addr + c * NUM_SUBJ], v,
                                       mask=msk)
            plsc.addupdate_scatter(acc, [addr + 3 * NUM_SUBJ], ones, mask=msk)
            return 0
        lax.fori_loop(0, GROUPS, group_body, 0)
        return 0

    lax.fori_loop(0, TRIPS, trip_body, 0)
    pltpu.sync_copy(acc, out_ref.at[wid])


def _tc_stats(x_ref, out_ref):
    x = x_ref[...]
    t = jnp.sum(x, axis=0, keepdims=True)
    counts = t[:, 3 * NUM_SUBJ:4 * NUM_SUBJ]
    present = (counts > 0).astype(jnp.float32)
    m = jnp.sum(present)
    safe_counts = jnp.maximum(counts, 1.0)
    targets = (400.0, 25.0, 0.05)
    loss = jnp.float32(0.0)
    for c in range(3):
        s_c = t[:, c * NUM_SUBJ:(c + 1) * NUM_SUBJ]
        mean_c = s_c / safe_counts
        mom_c = jnp.sum(mean_c * present) / m
        var_c = jnp.sum(present * (mean_c - mom_c) ** 2) / (m - 1.0)
        std_c = jnp.sqrt(var_c)
        loss = loss + jnp.maximum(targets[c] - std_c, 0.0)
    loss = jnp.where(m < 2.0, jnp.float32(0.0), loss)
    out_ref[0, 0] = loss


def kernel(params, subject_idx):
    params = params.astype(jnp.float32)
    sidx = subject_idx.astype(jnp.int32)

    mesh = plsc.VectorSubcoreMesh(core_axis_name="c", subcore_axis_name="s")
    partials = pl.kernel(
        _sc_segment_partials,
        out_type=jax.ShapeDtypeStruct((NUM_TILES, ACC_WORDS), jnp.float32),
        mesh=mesh,
        compiler_params=pltpu.CompilerParams(needs_layout_passes=False),
        scratch_types=[
            pltpu.VMEM((CHUNK, 3), jnp.float32),
            pltpu.VMEM((CHUNK,), jnp.int32),
            pltpu.VMEM((CHUNK,), jnp.int32),
            pltpu.VMEM((ACC_WORDS,), jnp.float32),
            pltpu.SemaphoreType.DMA,
            pltpu.SemaphoreType.DMA,
        ],
    )(params, sidx)

    stacked = partials.reshape(NUM_TILES * LANES, 4 * NUM_SUBJ)
    loss = pl.pallas_call(
        _tc_stats,
        out_shape=jax.ShapeDtypeStruct((1, 1), jnp.float32),
        out_specs=pl.BlockSpec(memory_space=pltpu.SMEM),
    )(stacked)
    return loss[0, 0]


# speedup vs baseline: 95.2732x; 10.4839x over previous
import jax
import jax.numpy as jnp
from jax import lax
from jax.experimental import pallas as pl
from jax.experimental.pallas import tpu as pltpu
from jax.experimental.pallas import tpu_sc as plsc

N_ROWS = 4_000_000
NUM_SUBJ = 64
LANES = 16
CHUNK = 4096
GROUPS = CHUNK // LANES
NUM_TILES = 32
NUM_CHUNKS = (N_ROWS + CHUNK - 1) // CHUNK
TRIPS = (NUM_CHUNKS + NUM_TILES - 1) // NUM_TILES
ACC_WORDS = LANES * 4 * NUM_SUBJ


def _sc_segment_partials(params_ref, sidx_ref, out_ref, par_buf, idx_buf,
                         acc, sem_p, sem_i):
    wid = lax.axis_index("s") * 2 + lax.axis_index("c")
    lane = lax.iota(jnp.int32, LANES)
    ones = jnp.ones((LANES,), jnp.float32)

    def zero_body(i, _):
        acc[pl.ds(i * LANES, LANES)] = jnp.zeros((LANES,), jnp.float32)
        return 0
    lax.fori_loop(0, ACC_WORDS // LANES, zero_body, 0)

    def trip_body(k, _):
        ci = wid + k * NUM_TILES
        nominal = ci * CHUNK
        start = jnp.minimum(nominal, N_ROWS - CHUNK)
        delta = nominal - start
        cps = [
            pltpu.async_copy(params_ref.at[pl.ds(c, 1), pl.ds(start, CHUNK)],
                             par_buf.at[pl.ds(c, 1), :], sem_p)
            for c in range(3)
        ]
        cp_i = pltpu.async_copy(sidx_ref.at[pl.ds(start, CHUNK)], idx_buf,
                                sem_i)
        for cp in cps:
            cp.wait()
        cp_i.wait()

        def group_body(g, _):
            base = g * LANES
            rows = base + lane
            msk = rows >= delta
            subj = idx_buf[pl.ds(base, LANES)]
            addr = lane * 256 + subj
            for c in range(3):
                v = par_buf[c, pl.ds(base, LANES)]
                plsc.addupdate_scatter(acc, [addr + c * NUM_SUBJ], v,
                                       mask=msk)
            plsc.addupdate_scatter(acc, [addr + 3 * NUM_SUBJ], ones, mask=msk)
            return 0
        lax.fori_loop(0, GROUPS, group_body, 0)
        return 0

    lax.fori_loop(0, TRIPS, trip_body, 0)
    pltpu.sync_copy(acc, out_ref.at[wid])


def _tc_stats(x_ref, out_ref):
    x = x_ref[...]
    t = jnp.sum(x, axis=0, keepdims=True)
    counts = t[:, 3 * NUM_SUBJ:4 * NUM_SUBJ]
    present = (counts > 0).astype(jnp.float32)
    m = jnp.sum(present)
    safe_counts = jnp.maximum(counts, 1.0)
    targets = (400.0, 25.0, 0.05)
    loss = jnp.float32(0.0)
    for c in range(3):
        s_c = t[:, c * NUM_SUBJ:(c + 1) * NUM_SUBJ]
        mean_c = s_c / safe_counts
        mom_c = jnp.sum(mean_c * present) / m
        var_c = jnp.sum(present * (mean_c - mom_c) ** 2) / (m - 1.0)
        std_c = jnp.sqrt(var_c)
        loss = loss + jnp.maximum(targets[c] - std_c, 0.0)
    loss = jnp.where(m < 2.0, jnp.float32(0.0), loss)
    out_ref[0, 0] = loss


def kernel(params, subject_idx):
    params3 = params.astype(jnp.float32).T
    sidx = subject_idx.astype(jnp.int32)

    mesh = plsc.VectorSubcoreMesh(core_axis_name="c", subcore_axis_name="s")
    partials = pl.kernel(
        _sc_segment_partials,
        out_type=jax.ShapeDtypeStruct((NUM_TILES, ACC_WORDS), jnp.float32),
        mesh=mesh,
        compiler_params=pltpu.CompilerParams(needs_layout_passes=False),
        scratch_types=[
            pltpu.VMEM((3, CHUNK), jnp.float32),
            pltpu.VMEM((CHUNK,), jnp.int32),
            pltpu.VMEM((ACC_WORDS,), jnp.float32),
            pltpu.SemaphoreType.DMA,
            pltpu.SemaphoreType.DMA,
        ],
    )(params3, sidx)

    stacked = partials.reshape(NUM_TILES * LANES, 4 * NUM_SUBJ)
    loss = pl.pallas_call(
        _tc_stats,
        out_shape=jax.ShapeDtypeStruct((1, 1), jnp.float32),
        out_specs=pl.BlockSpec(memory_space=pltpu.SMEM),
    )(stacked)
    return loss[0, 0]


# speedup vs baseline: 182.9067x; 1.9198x over previous
import jax
import jax.numpy as jnp
from jax import lax
from jax.experimental import pallas as pl
from jax.experimental.pallas import tpu as pltpu
from jax.experimental.pallas import tpu_sc as plsc

N_ROWS = 4_000_000
NUM_SUBJ = 64
LANES = 16
CHUNK = 8192
GROUPS = CHUNK // LANES
NUM_TILES = 32
NUM_CHUNKS = (N_ROWS + CHUNK - 1) // CHUNK
TRIPS = (NUM_CHUNKS + NUM_TILES - 1) // NUM_TILES
ACC_WORDS = LANES * 4 * NUM_SUBJ


def _sc_segment_partials(params_ref, sidx_ref, out_ref,
                         par0, par1, idx0, idx1,
                         acc, sp0, sp1, si0, si1):
    wid = lax.axis_index("s") * 2 + lax.axis_index("c")
    lane = lax.iota(jnp.int32, LANES)
    lane256 = lane * 256
    ones = jnp.ones((LANES,), jnp.float32)
    pbufs = (par0, par1)
    ibufs = (idx0, idx1)
    psems = (sp0, sp1)
    isems = (si0, si1)

    def zero_body(i, _):
        acc[pl.ds(i * LANES, LANES)] = jnp.zeros((LANES,), jnp.float32)
        return 0
    lax.fori_loop(0, ACC_WORDS // LANES, zero_body, 0)

    def chunk_start(k):
        ci = wid + k * NUM_TILES
        nominal = ci * CHUNK
        start = jnp.minimum(nominal, N_ROWS - CHUNK)
        return start, nominal - start

    def issue(k):
        start, _ = chunk_start(k)
        b = k % 2
        cps = [
            pltpu.async_copy(params_ref.at[pl.ds(c, 1), pl.ds(start, CHUNK)],
                             pbufs[b].at[pl.ds(c, 1), :], psems[b])
            for c in range(3)
        ]
        cps.append(pltpu.async_copy(sidx_ref.at[pl.ds(start, CHUNK)],
                                    ibufs[b], isems[b]))
        return cps

    def process(k):
        b = k % 2
        par_buf, idx_buf = pbufs[b], ibufs[b]
        if k < TRIPS - 1:
            @plsc.parallel_loop(0, CHUNK, LANES, unroll=4)
            def _(base):
                subj = idx_buf[pl.ds(base, LANES)]
                addr = lane256 + subj
                for c in range(3):
                    v = par_buf[c, pl.ds(base, LANES)]
                    plsc.addupdate_scatter(acc, [addr + c * NUM_SUBJ], v)
                plsc.addupdate_scatter(acc, [addr + 3 * NUM_SUBJ], ones)
        else:
            _, delta = chunk_start(k)

            @plsc.parallel_loop(0, CHUNK, LANES, unroll=4)
            def _(base):
                msk = (base + lane) >= delta
                subj = idx_buf[pl.ds(base, LANES)]
                addr = lane256 + subj
                for c in range(3):
                    v = par_buf[c, pl.ds(base, LANES)]
                    plsc.addupdate_scatter(acc, [addr + c * NUM_SUBJ], v,
                                           mask=msk)
                plsc.addupdate_scatter(acc, [addr + 3 * NUM_SUBJ], ones,
                                       mask=msk)

    cur = issue(0)
    for k in range(TRIPS):
        nxt = issue(k + 1) if k + 1 < TRIPS else None
        for cp in cur:
            cp.wait()
        process(k)
        cur = nxt

    pltpu.sync_copy(acc, out_ref.at[wid])


def _tc_stats(x_ref, out_ref):
    x = x_ref[...]
    t = jnp.sum(x, axis=0, keepdims=True)
    counts = t[:, 3 * NUM_SUBJ:4 * NUM_SUBJ]
    present = (counts > 0).astype(jnp.float32)
    m = jnp.sum(present)
    safe_counts = jnp.maximum(counts, 1.0)
    targets = (400.0, 25.0, 0.05)
    loss = jnp.float32(0.0)
    for c in range(3):
        s_c = t[:, c * NUM_SUBJ:(c + 1) * NUM_SUBJ]
        mean_c = s_c / safe_counts
        mom_c = jnp.sum(mean_c * present) / m
        var_c = jnp.sum(present * (mean_c - mom_c) ** 2) / (m - 1.0)
        std_c = jnp.sqrt(var_c)
        loss = loss + jnp.maximum(targets[c] - std_c, 0.0)
    loss = jnp.where(m < 2.0, jnp.float32(0.0), loss)
    out_ref[0, 0] = loss


def kernel(params, subject_idx):
    params3 = params.astype(jnp.float32).T
    sidx = subject_idx.astype(jnp.int32)

    mesh = plsc.VectorSubcoreMesh(core_axis_name="c", subcore_axis_name="s")
    partials = pl.kernel(
        _sc_segment_partials,
        out_type=jax.ShapeDtypeStruct((NUM_TILES, ACC_WORDS), jnp.float32),
        mesh=mesh,
        compiler_params=pltpu.CompilerParams(needs_layout_passes=False),
        scratch_types=[
            pltpu.VMEM((3, CHUNK), jnp.float32),
            pltpu.VMEM((3, CHUNK), jnp.float32),
            pltpu.VMEM((CHUNK,), jnp.int32),
            pltpu.VMEM((CHUNK,), jnp.int32),
            pltpu.VMEM((ACC_WORDS,), jnp.float32),
            pltpu.SemaphoreType.DMA,
            pltpu.SemaphoreType.DMA,
            pltpu.SemaphoreType.DMA,
            pltpu.SemaphoreType.DMA,
        ],
    )(params3, sidx)

    stacked = partials.reshape(NUM_TILES * LANES, 4 * NUM_SUBJ)
    loss = pl.pallas_call(
        _tc_stats,
        out_shape=jax.ShapeDtypeStruct((1, 1), jnp.float32),
        out_specs=pl.BlockSpec(memory_space=pltpu.SMEM),
    )(stacked)
    return loss[0, 0]
